# baseline (speedup 1.0000x reference)
import jax
import jax.numpy as jnp
from jax import lax
from jax.experimental import pallas as pl
from jax.experimental.pallas import tpu as pltpu
from jax.experimental.pallas import tpu_sc as plsc

B = 1024
N = 32
DIM_H = 128
DIM_K = 2
NUM_NODE_CLASSES = 42
NUM_EDGE_CLASSES = 6
NUM_H_CLASSES = 6
NUM_CHARGE_CLASSES = 6
NUM_RING_CLASSES = 3
NUM_AROM_CLASSES = 3
NUM_CHIR_CLASSES = 5

GRID = 64
RE = (B * N * N * DIM_K) // GRID

_SC_CORES = 2
_SC_SUBCORES = 16
_NW = _SC_CORES * _SC_SUBCORES
_BPW = (B * N) // _NW
_CH = 128


def _edge_body(eidx_ref, e_tab_ref, edge_out_ref):
    idx8 = eidx_ref[0]
    eidx = jnp.concatenate(
        [jnp.broadcast_to(idx8[s:s + 1, :], (16, RE // 8)) for s in range(8)],
        axis=1)
    e_iota = jax.lax.broadcasted_iota(jnp.int32, (16, RE), 0)
    e_onehot_t = (eidx == e_iota).astype(jnp.float32)
    edge_out_ref[...] = jax.lax.dot_general(
        e_onehot_t, e_tab_ref[...], (((0,), (0,)), ((), ())),
        preferred_element_type=jnp.float32)


def _node_sc_body(ia_ref, ib_ref, ic_ref, tab_ref, out_ref,
                  idx_a, idx_b, idx_c, rows_a, rows_b, rows_c, acc,
                  sem_a, sem_b, sem_c):
    wid = lax.axis_index("s") * _SC_CORES + lax.axis_index("c")
    base = wid * _BPW

    @pl.loop(0, _BPW // _CH)
    def _chunk(ch):
        off = base + ch * _CH
        pltpu.sync_copy(ia_ref.at[pl.ds(off, _CH)], idx_a)
        pltpu.sync_copy(ib_ref.at[pl.ds(off, _CH)], idx_b)
        pltpu.sync_copy(ic_ref.at[pl.ds(off, _CH)], idx_c)
        ca = pltpu.async_copy(tab_ref.at[idx_a], rows_a, sem_a)
        cb = pltpu.async_copy(tab_ref.at[idx_b], rows_b, sem_b)
        cc = pltpu.async_copy(tab_ref.at[idx_c], rows_c, sem_c)
        ca.wait()
        cb.wait()
        cc.wait()

        @pl.loop(0, _CH)
        def _row(i):
            for j in range(0, DIM_H, 16):
                acc[i, pl.ds(j, 16)] = (rows_a[i, pl.ds(j, 16)]
                                        + rows_b[i, pl.ds(j, 16)]
                                        + rows_c[i, pl.ds(j, 16)])

        pltpu.sync_copy(acc, out_ref.at[pl.ds(off, _CH)])


def kernel(node_inds, adj_mat_inds, init_hydrogens, init_charge,
           init_is_in_ring, init_is_aromatic, init_chirality,
           n_emb, e_emb, h_emb, charge_emb, ring_emb, arom_emb, chir_emb):
    comb_hc = (h_emb[:, None, :] + charge_emb[None, :, :]).reshape(-1, DIM_H)
    comb_rac = (ring_emb[:, None, None, :] + arom_emb[None, :, None, :]
                + chir_emb[None, None, :, :]).reshape(-1, DIM_H)
    tab_s = jnp.concatenate([n_emb, comb_hc, comb_rac], axis=0)
    off_hc = NUM_NODE_CLASSES
    off_rac = off_hc + NUM_H_CLASSES * NUM_CHARGE_CLASSES
    idx_a_h = node_inds.reshape(-1).astype(jnp.int32)
    idx_b_h = (off_hc + init_hydrogens.reshape(-1) * NUM_CHARGE_CLASSES
               + init_charge.reshape(-1)).astype(jnp.int32)
    idx_c_h = (off_rac
               + init_is_in_ring.reshape(-1) * (NUM_AROM_CLASSES * NUM_CHIR_CLASSES)
               + init_is_aromatic.reshape(-1) * NUM_CHIR_CLASSES
               + init_chirality.reshape(-1)).astype(jnp.int32)

    node_out = pl.kernel(
        _node_sc_body,
        out_type=jax.ShapeDtypeStruct((B * N, DIM_H), jnp.float32),
        mesh=plsc.VectorSubcoreMesh(core_axis_name="c", subcore_axis_name="s"),
        scratch_types=[
            pltpu.VMEM((_CH,), jnp.int32),
            pltpu.VMEM((_CH,), jnp.int32),
            pltpu.VMEM((_CH,), jnp.int32),
            pltpu.VMEM((_CH, DIM_H), jnp.float32),
            pltpu.VMEM((_CH, DIM_H), jnp.float32),
            pltpu.VMEM((_CH, DIM_H), jnp.float32),
            pltpu.VMEM((_CH, DIM_H), jnp.float32),
            pltpu.SemaphoreType.DMA,
            pltpu.SemaphoreType.DMA,
            pltpu.SemaphoreType.DMA,
        ],
    )(idx_a_h, idx_b_h, idx_c_h, tab_s)

    eidx2 = (adj_mat_inds.reshape(-1, 1) * 2
             + jnp.arange(2, dtype=adj_mat_inds.dtype))
    eidx2 = eidx2.reshape(GRID, 8, RE // 8)

    e_tab = jnp.zeros((16, DIM_H), jnp.float32).at[:2 * NUM_EDGE_CLASSES].set(
        e_emb.reshape(NUM_EDGE_CLASSES, DIM_H, DIM_K)
             .transpose(0, 2, 1).reshape(2 * NUM_EDGE_CLASSES, DIM_H))

    edge_out = pl.pallas_call(
        _edge_body,
        grid=(GRID,),
        in_specs=[
            pl.BlockSpec((1, 8, RE // 8), lambda i: (i, 0, 0)),
            pl.BlockSpec((16, DIM_H), lambda i: (0, 0)),
        ],
        out_specs=pl.BlockSpec((RE, DIM_H), lambda i: (i, 0)),
        out_shape=jax.ShapeDtypeStruct((B * N * N * DIM_K, DIM_H), jnp.float32),
    )(eidx2, e_tab)

    edge5 = edge_out.reshape(B, N, N, DIM_K, DIM_H).swapaxes(-1, -2)
    return (node_out.reshape(B, N, DIM_H), edge5)


# speedup vs baseline: 1.2150x; 1.2150x over previous
import jax
import jax.numpy as jnp
from jax.experimental import pallas as pl
from jax.experimental.pallas import tpu as pltpu

B = 1024
N = 32
DIM_H = 128
DIM_K = 2
NUM_NODE_CLASSES = 42
NUM_EDGE_CLASSES = 6
NUM_H_CLASSES = 6
NUM_CHARGE_CLASSES = 6
NUM_RING_CLASSES = 3
NUM_AROM_CLASSES = 3
NUM_CHIR_CLASSES = 5

_NODE_SIZES = (NUM_NODE_CLASSES, NUM_H_CLASSES, NUM_CHARGE_CLASSES,
               NUM_RING_CLASSES, NUM_AROM_CLASSES, NUM_CHIR_CLASSES)
_NODE_OFFSETS = tuple(sum(_NODE_SIZES[:i]) for i in range(len(_NODE_SIZES)))

GRID = 64
RE = (B * N * N * DIM_K) // GRID
RN = (B * N) // GRID


def _fused_body(eidx_ref, nidx_ref, e_tab_ref, n_tab_ref,
                edge_out_ref, node_out_ref):
    idx8 = eidx_ref[0]
    eidx = jnp.concatenate(
        [jnp.broadcast_to(idx8[s:s + 1, :], (16, RE // 8)) for s in range(8)],
        axis=1)
    e_iota = jax.lax.broadcasted_iota(jnp.int32, (16, RE), 0)
    e_onehot_t = (eidx == e_iota).astype(jnp.float32)
    edge_out_ref[...] = jax.lax.dot_general(
        e_onehot_t, e_tab_ref[...], (((0,), (0,)), ((), ())),
        preferred_element_type=jnp.float32)

    n_iota = jax.lax.broadcasted_iota(jnp.int32, (RN, 128), 1)
    m = jnp.zeros((RN, 128), jnp.float32)
    for t, off in enumerate(_NODE_OFFSETS):
        idx_t = nidx_ref[0, t, :]
        m = m + (idx_t[:, None] + off == n_iota).astype(jnp.float32)
    node_out_ref[...] = jnp.dot(m, n_tab_ref[...],
                                preferred_element_type=jnp.float32)


def kernel(node_inds, adj_mat_inds, init_hydrogens, init_charge,
           init_is_in_ring, init_is_aromatic, init_chirality,
           n_emb, e_emb, h_emb, charge_emb, ring_emb, arom_emb, chir_emb):
    eidx2 = (adj_mat_inds.reshape(-1, 1) * 2
             + jnp.arange(2, dtype=adj_mat_inds.dtype))
    eidx2 = eidx2.reshape(GRID, 8, RE // 8)
    nidx = jnp.stack([a.reshape(GRID, RN) for a in
                      (node_inds, init_hydrogens, init_charge,
                       init_is_in_ring, init_is_aromatic, init_chirality)],
                     axis=1)

    e_tab = jnp.zeros((16, DIM_H), jnp.float32).at[:2 * NUM_EDGE_CLASSES].set(
        e_emb.reshape(NUM_EDGE_CLASSES, DIM_H, DIM_K)
             .transpose(0, 2, 1).reshape(2 * NUM_EDGE_CLASSES, DIM_H))
    n_tab = jnp.zeros((128, DIM_H), jnp.float32)
    for tab, off in zip((n_emb, h_emb, charge_emb, ring_emb, arom_emb, chir_emb),
                        _NODE_OFFSETS):
        n_tab = n_tab.at[off:off + tab.shape[0]].set(tab)

    edge_out, node_out = pl.pallas_call(
        _fused_body,
        grid=(GRID,),
        in_specs=[
            pl.BlockSpec((1, 8, RE // 8), lambda i: (i, 0, 0)),
            pl.BlockSpec((1, 6, RN), lambda i: (i, 0, 0)),
            pl.BlockSpec((16, DIM_H), lambda i: (0, 0)),
            pl.BlockSpec((128, DIM_H), lambda i: (0, 0)),
        ],
        out_specs=[
            pl.BlockSpec((RE, DIM_H), lambda i: (i, 0)),
            pl.BlockSpec((RN, DIM_H), lambda i: (i, 0)),
        ],
        out_shape=[
            jax.ShapeDtypeStruct((B * N * N * DIM_K, DIM_H), jnp.float32),
            jax.ShapeDtypeStruct((B * N, DIM_H), jnp.float32),
        ],
    )(eidx2, nidx, e_tab, n_tab)

    edge5 = edge_out.reshape(B, N, N, DIM_K, DIM_H).swapaxes(-1, -2)
    return (node_out.reshape(B, N, DIM_H), edge5)
